# baseline (speedup 1.0000x reference)
import functools

import numpy as np
import jax
import jax.numpy as jnp
from jax import lax
from jax.experimental import pallas as pl
from jax.experimental.pallas import tpu as pltpu
from jax.experimental.pallas import tpu_sc as plsc

_N_SEG = 64
_NC, _NS = 2, 16
_NW = _NC * _NS


def _segment_starts(nT: int) -> np.ndarray:
    t_vec = np.linspace(1, nT, _N_SEG + 1)
    return np.asarray([int(round(x)) - 1 for x in t_vec[:-1]], dtype=np.int32)


@functools.lru_cache(maxsize=None)
def _build(B: int, nT: int, D: int):
    idx = _segment_starts(nT)
    gidx = (np.arange(B, dtype=np.int64)[:, None] * nT + idx[None, :])
    gidx = gidx.reshape(-1).astype(np.int32)
    n_rows = gidx.size
    assert n_rows % _NW == 0
    rpw = n_rows // _NW

    assert _N_SEG % rpw == 0
    wpb = _N_SEG // rpw

    mesh = plsc.VectorSubcoreMesh(
        core_axis_name="c", subcore_axis_name="s",
        num_cores=_NC, num_subcores=_NS)

    nck = 4
    csz = rpw // nck

    @functools.partial(
        pl.kernel, mesh=mesh,
        out_type=jax.ShapeDtypeStruct((B, _N_SEG, D), jnp.float32),
        scratch_types=[
            pltpu.VMEM((nck, csz), jnp.int32),
            pltpu.VMEM((nck, csz, D), jnp.float32),
            pltpu.SemaphoreType.DMA,
            pltpu.SemaphoreType.DMA,
        ],
    )
    def gather_rows(table_hbm, idx_hbm, out_hbm, idx_v, rows_v, gsem, ssem):
        wid = lax.axis_index("s") * _NC + lax.axis_index("c")
        b = wid // wpb
        j0 = (wid % wpb) * rpw
        pltpu.sync_copy(idx_hbm.at[wid], idx_v)
        gathers = [
            pltpu.async_copy(table_hbm.at[idx_v.at[k]], rows_v.at[k], gsem)
            for k in range(nck)]
        scatters = []
        for k in range(nck):
            gathers[k].wait()
            scatters.append(pltpu.async_copy(
                rows_v.at[k], out_hbm.at[b, pl.ds(j0 + k * csz, csz)], ssem))
        for s in scatters:
            s.wait()

    return gather_rows, gidx.reshape(_NW, nck, csz)


def kernel(inp):
    B, nT, D = inp.shape
    gather_rows, gidx = _build(B, nT, D)
    return gather_rows(inp.reshape(B * nT, D), jnp.asarray(gidx))


# speedup vs baseline: 1.0259x; 1.0259x over previous
import functools

import numpy as np
import jax
import jax.numpy as jnp
from jax import lax
from jax.experimental import pallas as pl
from jax.experimental.pallas import tpu as pltpu
from jax.experimental.pallas import tpu_sc as plsc

_N_SEG = 64
_NC, _NS = 1, 16
_NW = _NC * _NS


def _segment_starts(nT: int) -> np.ndarray:
    t_vec = np.linspace(1, nT, _N_SEG + 1)
    return np.asarray([int(round(x)) - 1 for x in t_vec[:-1]], dtype=np.int32)


@functools.lru_cache(maxsize=None)
def _build(B: int, nT: int, D: int):
    idx = _segment_starts(nT)
    gidx = (np.arange(B, dtype=np.int64)[:, None] * nT + idx[None, :])
    gidx = gidx.reshape(-1).astype(np.int32)
    n_rows = gidx.size
    assert n_rows % _NW == 0
    rpw = n_rows // _NW

    assert _N_SEG % rpw == 0
    wpb = _N_SEG // rpw

    mesh = plsc.VectorSubcoreMesh(
        core_axis_name="c", subcore_axis_name="s",
        num_cores=_NC, num_subcores=_NS)

    nck = 4
    csz = rpw // nck

    @functools.partial(
        pl.kernel, mesh=mesh,
        out_type=jax.ShapeDtypeStruct((B, _N_SEG, D), jnp.float32),
        scratch_types=[
            pltpu.VMEM((nck, csz), jnp.int32),
            pltpu.VMEM((nck, csz, D), jnp.float32),
            pltpu.SemaphoreType.DMA,
            pltpu.SemaphoreType.DMA,
        ],
    )
    def gather_rows(table_hbm, idx_hbm, out_hbm, idx_v, rows_v, gsem, ssem):
        wid = lax.axis_index("s") * _NC + lax.axis_index("c")
        b = wid // wpb
        j0 = (wid % wpb) * rpw
        pltpu.sync_copy(idx_hbm.at[wid], idx_v)
        gathers = [
            pltpu.async_copy(table_hbm.at[idx_v.at[k]], rows_v.at[k], gsem)
            for k in range(nck)]
        scatters = []
        for k in range(nck):
            gathers[k].wait()
            scatters.append(pltpu.async_copy(
                rows_v.at[k], out_hbm.at[b, pl.ds(j0 + k * csz, csz)], ssem))
        for s in scatters:
            s.wait()

    return gather_rows, gidx.reshape(_NW, nck, csz)


def kernel(inp):
    B, nT, D = inp.shape
    gather_rows, gidx = _build(B, nT, D)
    return gather_rows(inp.reshape(B * nT, D), jnp.asarray(gidx))


# speedup vs baseline: 1.0276x; 1.0017x over previous
import functools

import numpy as np
import jax
import jax.numpy as jnp
from jax import lax
from jax.experimental import pallas as pl
from jax.experimental.pallas import tpu as pltpu
from jax.experimental.pallas import tpu_sc as plsc

_N_SEG = 64
_NC, _NS = 1, 16
_NW = _NC * _NS


def _segment_starts(nT: int) -> np.ndarray:
    t_vec = np.linspace(1, nT, _N_SEG + 1)
    return np.asarray([int(round(x)) - 1 for x in t_vec[:-1]], dtype=np.int32)


@functools.lru_cache(maxsize=None)
def _build(B: int, nT: int, D: int):
    idx = _segment_starts(nT)
    gidx = (np.arange(B, dtype=np.int64)[:, None] * nT + idx[None, :])
    gidx = gidx.reshape(-1).astype(np.int32)
    n_rows = gidx.size
    assert n_rows % _NW == 0
    rpw = n_rows // _NW

    assert _N_SEG % rpw == 0
    wpb = _N_SEG // rpw

    mesh = plsc.VectorSubcoreMesh(
        core_axis_name="c", subcore_axis_name="s",
        num_cores=_NC, num_subcores=_NS)

    nck = 4
    csz = rpw // nck

    @functools.partial(
        pl.kernel, mesh=mesh,
        compiler_params=pltpu.CompilerParams(skip_device_barrier=True),
        out_type=jax.ShapeDtypeStruct((B, _N_SEG, D), jnp.float32),
        scratch_types=[
            pltpu.VMEM((nck, csz), jnp.int32),
            pltpu.VMEM((nck, csz, D), jnp.float32),
            pltpu.SemaphoreType.DMA,
            pltpu.SemaphoreType.DMA,
        ],
    )
    def gather_rows(table_hbm, idx_hbm, out_hbm, idx_v, rows_v, gsem, ssem):
        wid = lax.axis_index("s") * _NC + lax.axis_index("c")
        b = wid // wpb
        j0 = (wid % wpb) * rpw
        pltpu.sync_copy(idx_hbm.at[wid], idx_v)
        gathers = [
            pltpu.async_copy(table_hbm.at[idx_v.at[k]], rows_v.at[k], gsem)
            for k in range(nck)]
        scatters = []
        for k in range(nck):
            gathers[k].wait()
            scatters.append(pltpu.async_copy(
                rows_v.at[k], out_hbm.at[b, pl.ds(j0 + k * csz, csz)], ssem))
        for s in scatters:
            s.wait()

    return gather_rows, gidx.reshape(_NW, nck, csz)


def kernel(inp):
    B, nT, D = inp.shape
    gather_rows, gidx = _build(B, nT, D)
    return gather_rows(inp.reshape(B * nT, D), jnp.asarray(gidx))
